# baseline (speedup 1.0000x reference)
import functools

import jax
import jax.numpy as jnp
from jax import lax
from jax.experimental import pallas as pl
from jax.experimental.pallas import tpu as pltpu
from jax.experimental.pallas import tpu_sc as plsc

_N = 10000
_E = 320000
_G = 256
_H = 128
_TILES = 16
_CORES = 2
_KB = 128
_NB = 157
_PER_TILE = _KB * _NB
_EPAD = _PER_TILE * _TILES
_ACC_ROWS = 10112
_ZR = 632
_DEG_ROWS = 10112
_R = 1000

_PREC = lax.Precision.HIGHEST


def _edge_body(t_hbm, src_hbm, dst_hbm, s_out, acc,
               idx_s0, idx_d0, idx_s1, idx_d1, idx_s2, idx_d2,
               rows0, rows1, rows2,
               semg0, semg1, semg2, sems0, sems1, sems2,
               semi0, semi1, semi2):
    c = lax.axis_index("c")
    s = lax.axis_index("s")
    zero16 = jnp.zeros((16,), jnp.float32)

    ISX = (idx_s0, idx_s1, idx_s2)
    IDX = (idx_d0, idx_d1, idx_d2)
    ROWS = (rows0, rows1, rows2)
    SEMG = (semg0, semg1, semg2)
    SEMS = (sems0, sems1, sems2)
    SEMI = (semi0, semi1, semi2)

    def zrow(r, carry):
        for j in range(8):
            rows0[r, pl.ds(j * 16, 16)] = zero16
        return carry

    lax.fori_loop(0, _KB, zrow, 0)
    zb = s * _ZR
    for j in range(4):
        pltpu.sync_copy(rows0, acc.at[pl.ds(zb + j * _KB, _KB)])
    pltpu.sync_copy(rows0.at[pl.ds(0, _ZR - 4 * _KB)],
                    acc.at[pl.ds(zb + 4 * _KB, _ZR - 4 * _KB)])
    plsc.subcore_barrier()

    coff = c * _N
    ebase = s * _PER_TILE

    def idxfire(b, k):
        base = ebase + b * _KB
        pltpu.async_copy(src_hbm.at[pl.ds(base, _KB)], ISX[k], SEMI[k])
        pltpu.async_copy(dst_hbm.at[pl.ds(base, _KB)], IDX[k], SEMI[k])

    def idxwait(k):
        pltpu.make_async_copy(src_hbm.at[pl.ds(0, _KB)], ISX[k],
                              SEMI[k]).wait()
        pltpu.make_async_copy(dst_hbm.at[pl.ds(0, _KB)], IDX[k],
                              SEMI[k]).wait()

    def coffadd(k):
        for j in range(_KB // 16):
            sl = pl.ds(j * 16, 16)
            ISX[k][sl] = ISX[k][sl] + coff

    def gfire(b, k):
        pltpu.async_copy(t_hbm.at[ISX[k]], ROWS[k], SEMG[k])

    def gwait(k):
        pltpu.make_async_copy(t_hbm.at[ISX[k]], ROWS[k], SEMG[k]).wait()

    def scfire(k):
        pltpu.async_copy(ROWS[k], acc.at[IDX[k]], SEMS[k], add=True)

    def scwait(k):
        pltpu.make_async_copy(ROWS[k], acc.at[IDX[k]], SEMS[k]).wait()

    def step(b, k, first, do_idx, do_g):
        kn = (k + 2) % 3
        k1 = (k + 1) % 3
        gwait(k)
        scfire(k)
        if not first:
            scwait(kn)
        if do_idx:
            idxfire(b + 2, kn)
        if do_g:
            idxwait(k1)
            coffadd(k1)
            gfire(b + 1, k1)

    idxfire(0, 0)
    idxwait(0)
    coffadd(0)
    gfire(0, 0)
    idxfire(1, 1)
    step(0, 0, first=True, do_idx=True, do_g=True)

    def triple(q, carry):
        b = 3 * q + 1
        step(b, 1, False, True, True)
        step(b + 1, 2, False, True, True)
        step(b + 2, 0, False, True, True)
        return carry

    lax.fori_loop(0, 50, triple, 0)
    step(151, 1, False, True, True)
    step(152, 2, False, True, True)
    step(153, 0, False, True, True)
    step(154, 1, False, True, True)
    step(155, 2, False, False, True)
    step(156, 0, False, False, False)
    scwait(0)

    plsc.subcore_barrier()
    ob = s * _ZR
    pltpu.sync_copy(acc.at[pl.ds(ob, _ZR)],
                    s_out.at[pl.ds(c * _ACC_ROWS + ob, _ZR)])


@functools.lru_cache(maxsize=None)
def _edge_kernel_fn():
    mesh = plsc.VectorSubcoreMesh(
        core_axis_name="c", subcore_axis_name="s",
        num_cores=_CORES, num_subcores=_TILES)
    return pl.kernel(
        _edge_body,
        out_type=jax.ShapeDtypeStruct((2 * _ACC_ROWS, _H), jnp.float32),
        mesh=mesh,
        scratch_types=[
            pltpu.VMEM_SHARED((_ACC_ROWS, _H), jnp.float32),
            pltpu.VMEM((_KB,), jnp.int32),
            pltpu.VMEM((_KB,), jnp.int32),
            pltpu.VMEM((_KB,), jnp.int32),
            pltpu.VMEM((_KB,), jnp.int32),
            pltpu.VMEM((_KB,), jnp.int32),
            pltpu.VMEM((_KB,), jnp.int32),
            pltpu.VMEM((_KB, _H), jnp.float32),
            pltpu.VMEM((_KB, _H), jnp.float32),
            pltpu.VMEM((_KB, _H), jnp.float32),
            pltpu.SemaphoreType.DMA,
            pltpu.SemaphoreType.DMA,
            pltpu.SemaphoreType.DMA,
            pltpu.SemaphoreType.DMA,
            pltpu.SemaphoreType.DMA,
            pltpu.SemaphoreType.DMA,
            pltpu.SemaphoreType.DMA,
            pltpu.SemaphoreType.DMA,
            pltpu.SemaphoreType.DMA,
        ])


def _edge_kernel(t, src, dst):
    return _edge_kernel_fn()(t, src, dst)


def _deg_body(dst_both, deg_out, acc, idx, ones, zbuf):
    c = lax.axis_index("c")
    s = lax.axis_index("s")
    zero16 = jnp.zeros((16,), jnp.float32)
    one16 = jnp.ones((16,), jnp.float32)

    def fill(q, carry):
        zbuf[pl.ds(q * 16, 16)] = zero16
        return carry

    lax.fori_loop(0, 40, fill, 0)
    for j in range(8):
        ones[pl.ds(j * 16, 16)] = one16
    pltpu.sync_copy(zbuf.at[pl.ds(0, 632)], acc.at[pl.ds(s * 632, 632)])
    plsc.subcore_barrier()

    ebase = c * _EPAD + s * _PER_TILE

    def body(b, carry):
        pltpu.sync_copy(dst_both.at[pl.ds(ebase + b * _KB, _KB)], idx)
        pltpu.sync_copy(ones, acc.at[idx], add=True)
        return carry

    lax.fori_loop(0, _NB, body, 0)
    plsc.subcore_barrier()
    pltpu.sync_copy(acc.at[pl.ds(s * 632, 632)], zbuf.at[pl.ds(0, 632)])
    pltpu.sync_copy(zbuf.at[pl.ds(0, 632)],
                    deg_out.at[pl.ds(c * _DEG_ROWS + s * 632, 632)])


@functools.lru_cache(maxsize=None)
def _deg_kernel_fn():
    mesh = plsc.VectorSubcoreMesh(
        core_axis_name="c", subcore_axis_name="s",
        num_cores=_CORES, num_subcores=_TILES)
    return pl.kernel(
        _deg_body,
        out_type=jax.ShapeDtypeStruct((_CORES * _DEG_ROWS,), jnp.float32),
        mesh=mesh,
        scratch_types=[
            pltpu.VMEM_SHARED((_DEG_ROWS,), jnp.float32),
            pltpu.VMEM((_KB,), jnp.int32),
            pltpu.VMEM((_KB,), jnp.float32),
            pltpu.VMEM((640,), jnp.float32),
        ])


def _deg_kernel(dst_both):
    return _deg_kernel_fn()(dst_both)


def _layer1(x, deg, W, b, interpret=False):
    def body(x_ref, deg_ref, w_ref, b_ref, t_ref, invs_ref):
        invs = lax.rsqrt(deg_ref[...] + 1.0)
        invs_ref[...] = invs
        g = jnp.dot(x_ref[...], w_ref[...],
                    preferred_element_type=jnp.float32, precision=_PREC)
        t_ref[...] = invs * (g + b_ref[...])

    return pl.pallas_call(
        body,
        grid=(_N // _R, 2),
        in_specs=[
            pl.BlockSpec((_R, 128), lambda i, h: (i, 0)),
            pl.BlockSpec((_R, 1), lambda i, h: (i, 0)),
            pl.BlockSpec((128, _H), lambda i, h: (0, h)),
            pl.BlockSpec((1, _H), lambda i, h: (0, h)),
        ],
        out_specs=[
            pl.BlockSpec((_R, _H), lambda i, h: (h * (_N // _R) + i, 0)),
            pl.BlockSpec((_R, 1), lambda i, h: (i, 0)),
        ],
        out_shape=[
            jax.ShapeDtypeStruct((2 * _N, _H), jnp.float32),
            jax.ShapeDtypeStruct((_N, 1), jnp.float32),
        ],
        interpret=interpret,
    )(x, deg, W, b.reshape(1, -1))


def _layer23(S3, t3, invs, W, b, interpret=False):
    def body(s_ref, t_ref, invs_ref, w_ref, b_ref, out_ref):
        invs = invs_ref[...]
        x0 = jnp.maximum(invs * (s_ref[0] + t_ref[0]), 0.0)
        x1 = jnp.maximum(invs * (s_ref[1] + t_ref[1]), 0.0)
        x = jnp.concatenate([x0, x1], axis=1)
        g = jnp.dot(x, w_ref[...],
                    preferred_element_type=jnp.float32, precision=_PREC)
        out_ref[...] = invs * (g + b_ref[...])

    return pl.pallas_call(
        body,
        grid=(_N // _R, 2),
        in_specs=[
            pl.BlockSpec((2, _R, 128), lambda i, h: (0, i, 0)),
            pl.BlockSpec((2, _R, 128), lambda i, h: (0, i, 0)),
            pl.BlockSpec((_R, 1), lambda i, h: (i, 0)),
            pl.BlockSpec((256, _H), lambda i, h: (0, h)),
            pl.BlockSpec((1, _H), lambda i, h: (0, h)),
        ],
        out_specs=pl.BlockSpec((_R, _H), lambda i, h: (h * (_N // _R) + i, 0)),
        out_shape=jax.ShapeDtypeStruct((2 * _N, _H), jnp.float32),
        interpret=interpret,
    )(S3, t3, invs, W, b.reshape(1, -1))


def _pool(S3, t3, invs, batch2, interpret=False):
    nblk = _N // _R

    def body(s_ref, t_ref, invs_ref, b_ref, out_ref, cnt_ref):
        i = pl.program_id(0)
        invs = invs_ref[...]
        x0 = jnp.maximum(invs * (s_ref[0] + t_ref[0]), 0.0)
        x1 = jnp.maximum(invs * (s_ref[1] + t_ref[1]), 0.0)
        h = jnp.concatenate([x0, x1], axis=1)
        ids = lax.broadcasted_iota(jnp.int32, (_R, _G), 1)
        oh = (b_ref[...] == ids).astype(jnp.float32)
        part = lax.dot_general(oh, h, (((0,), (0,)), ((), ())),
                               preferred_element_type=jnp.float32,
                               precision=_PREC)
        cpart = lax.dot_general(oh, jnp.ones((_R, 1), jnp.float32),
                                (((0,), (0,)), ((), ())),
                                preferred_element_type=jnp.float32,
                                precision=_PREC)

        @pl.when(i == 0)
        def _():
            out_ref[...] = jnp.zeros_like(out_ref)
            cnt_ref[...] = jnp.zeros_like(cnt_ref)

        out_ref[...] += part
        cnt_ref[...] += cpart

        @pl.when(i == nblk - 1)
        def _():
            out_ref[...] = out_ref[...] / jnp.maximum(cnt_ref[...], 1.0)

    return pl.pallas_call(
        body,
        grid=(nblk,),
        in_specs=[
            pl.BlockSpec((2, _R, 128), lambda i: (0, i, 0)),
            pl.BlockSpec((2, _R, 128), lambda i: (0, i, 0)),
            pl.BlockSpec((_R, 1), lambda i: (i, 0)),
            pl.BlockSpec((_R, 1), lambda i: (i, 0)),
        ],
        out_specs=pl.BlockSpec((_G, 256), lambda i: (0, 0)),
        out_shape=jax.ShapeDtypeStruct((_G, 256), jnp.float32),
        scratch_shapes=[pltpu.VMEM((_G, 1), jnp.float32)],
        interpret=interpret,
    )(S3, t3, invs, batch2)


def _classifier(p1, p2, W1a, W1b, b1, W2, b2, W3, b3, interpret=False):
    def body(p1_ref, p2_ref, w1a, w1b, b1_ref, w2_ref, b2_ref, w3_ref, b3_ref,
             out_ref):
        z = jnp.dot(p1_ref[...], w1a[...],
                    preferred_element_type=jnp.float32, precision=_PREC)
        z += jnp.dot(p2_ref[...], w1b[...],
                     preferred_element_type=jnp.float32, precision=_PREC)
        z = jnp.maximum(z + b1_ref[...], 0.0)
        z = jnp.maximum(
            jnp.dot(z, w2_ref[...], preferred_element_type=jnp.float32,
                    precision=_PREC) + b2_ref[...], 0.0)
        out_ref[...] = jnp.dot(
            z, w3_ref[...], preferred_element_type=jnp.float32,
            precision=_PREC) + b3_ref[...]

    return pl.pallas_call(
        body,
        out_shape=jax.ShapeDtypeStruct((_G, 86), jnp.float32),
        interpret=interpret,
    )(p1, p2, W1a, W1b, b1.reshape(1, -1), W2, b2.reshape(1, -1),
      W3, b3.reshape(1, -1))


def kernel(drug1_x, drug1_edge_index, drug1_batch,
           drug2_x, drug2_edge_index, drug2_batch,
           We1, be1, We2, be2, We3, be3,
           Wc1, bc1, Wc2, bc2, Wc3, bc3):
    pad = _EPAD - _E

    def prep(ei):
        src = jnp.concatenate([ei[0], jnp.zeros((pad,), jnp.int32)])
        dst = jnp.concatenate([ei[1], jnp.full((pad,), _N, jnp.int32)])
        return src, dst

    src1, dst1 = prep(drug1_edge_index)
    src2, dst2 = prep(drug2_edge_index)

    deg_flat = _deg_kernel(jnp.concatenate([dst1, dst2]))
    deg1 = deg_flat[0:_N].reshape(_N, 1)
    deg2 = deg_flat[_DEG_ROWS:_DEG_ROWS + _N].reshape(_N, 1)

    def encode(x, src, dst, deg, batch):
        t, invs = _layer1(x, deg, We1, be1)
        for W, b in ((We2, be2), (We3, be3)):
            S = _edge_kernel(t, src, dst)
            t = _layer23(S.reshape(2, _ACC_ROWS, _H), t.reshape(2, _N, _H),
                         invs, W, b)
        S = _edge_kernel(t, src, dst)
        return _pool(S.reshape(2, _ACC_ROWS, _H), t.reshape(2, _N, _H),
                     invs, batch.reshape(_N, 1))

    p1 = encode(drug1_x, src1, dst1, deg1, drug1_batch)
    p2 = encode(drug2_x, src2, dst2, deg2, drug2_batch)
    return _classifier(p1, p2, Wc1[:256], Wc1[256:], bc1, Wc2, bc2, Wc3, bc3)


# speedup vs baseline: 1.0508x; 1.0508x over previous
import functools

import jax
import jax.numpy as jnp
from jax import lax
from jax.experimental import pallas as pl
from jax.experimental.pallas import tpu as pltpu
from jax.experimental.pallas import tpu_sc as plsc

_N = 10000
_E = 320000
_G = 256
_H = 128
_TILES = 16
_CORES = 2
_KB = 128
_NB = 157
_PER_TILE = _KB * _NB
_EPAD = _PER_TILE * _TILES
_ACC_ROWS = 10112
_ZR = 632
_DEG_ROWS = 10112
_R = 1000

_PREC = lax.Precision.HIGHEST


def _edge_body(t_hbm, src_hbm, dst_hbm, s_out, acc,
               idx_s0, idx_d0, idx_s1, idx_d1, idx_s2, idx_d2,
               rows0, rows1, rows2,
               semg0, semg1, semg2, sems0, sems1, sems2):
    c = lax.axis_index("c")
    s = lax.axis_index("s")
    zero16 = jnp.zeros((16,), jnp.float32)

    ISX = (idx_s0, idx_s1, idx_s2)
    IDX = (idx_d0, idx_d1, idx_d2)
    ROWS = (rows0, rows1, rows2)
    SEMG = (semg0, semg1, semg2)
    SEMS = (sems0, sems1, sems2)

    def zrow(r, carry):
        for j in range(8):
            rows0[r, pl.ds(j * 16, 16)] = zero16
        return carry

    lax.fori_loop(0, _KB, zrow, 0)
    zb = s * _ZR
    for j in range(4):
        pltpu.sync_copy(rows0, acc.at[pl.ds(zb + j * _KB, _KB)])
    pltpu.sync_copy(rows0.at[pl.ds(0, _ZR - 4 * _KB)],
                    acc.at[pl.ds(zb + 4 * _KB, _ZR - 4 * _KB)])
    plsc.subcore_barrier()

    coff = c * _N
    ebase = s * _PER_TILE

    def load_idx(b, k):
        base = ebase + b * _KB
        pltpu.sync_copy(src_hbm.at[pl.ds(base, _KB)], ISX[k])
        pltpu.sync_copy(dst_hbm.at[pl.ds(base, _KB)], IDX[k])
        for j in range(_KB // 16):
            sl = pl.ds(j * 16, 16)
            ISX[k][sl] = ISX[k][sl] + coff

    def gfire(b, k):
        pltpu.async_copy(t_hbm.at[ISX[k]], ROWS[k], SEMG[k])

    def gwait(k):
        pltpu.make_async_copy(t_hbm.at[ISX[k]], ROWS[k], SEMG[k]).wait()

    def scfire(k):
        pltpu.async_copy(ROWS[k], acc.at[IDX[k]], SEMS[k], add=True)

    def scwait(k):
        pltpu.make_async_copy(ROWS[k], acc.at[IDX[k]], SEMS[k]).wait()

    def step(b, k, first, last):
        kn = (k + 2) % 3
        gwait(k)
        scfire(k)
        if not first:
            scwait(kn)
        if not last:
            load_idx(b + 2, kn)
            gfire(b + 2, kn)

    load_idx(0, 0)
    gfire(0, 0)
    load_idx(1, 1)
    gfire(1, 1)
    step(0, 0, first=True, last=False)

    def triple(q, carry):
        b = 3 * q + 1
        step(b, 1, first=False, last=False)
        step(b + 1, 2, first=False, last=False)
        step(b + 2, 0, first=False, last=False)
        return carry

    lax.fori_loop(0, 50, triple, 0)
    step(151, 1, first=False, last=False)
    step(152, 2, first=False, last=False)
    step(153, 0, first=False, last=False)
    step(154, 1, first=False, last=False)
    step(155, 2, first=False, last=True)
    step(156, 0, first=False, last=True)
    scwait(0)

    plsc.subcore_barrier()
    ob = s * _ZR
    pltpu.sync_copy(acc.at[pl.ds(ob, _ZR)],
                    s_out.at[pl.ds(c * _ACC_ROWS + ob, _ZR)])


@functools.lru_cache(maxsize=None)
def _edge_kernel_fn():
    mesh = plsc.VectorSubcoreMesh(
        core_axis_name="c", subcore_axis_name="s",
        num_cores=_CORES, num_subcores=_TILES)
    return pl.kernel(
        _edge_body,
        out_type=jax.ShapeDtypeStruct((2 * _ACC_ROWS, _H), jnp.float32),
        mesh=mesh,
        scratch_types=[
            pltpu.VMEM_SHARED((_ACC_ROWS, _H), jnp.float32),
            pltpu.VMEM((_KB,), jnp.int32),
            pltpu.VMEM((_KB,), jnp.int32),
            pltpu.VMEM((_KB,), jnp.int32),
            pltpu.VMEM((_KB,), jnp.int32),
            pltpu.VMEM((_KB,), jnp.int32),
            pltpu.VMEM((_KB,), jnp.int32),
            pltpu.VMEM((_KB, _H), jnp.float32),
            pltpu.VMEM((_KB, _H), jnp.float32),
            pltpu.VMEM((_KB, _H), jnp.float32),
            pltpu.SemaphoreType.DMA,
            pltpu.SemaphoreType.DMA,
            pltpu.SemaphoreType.DMA,
            pltpu.SemaphoreType.DMA,
            pltpu.SemaphoreType.DMA,
            pltpu.SemaphoreType.DMA,
        ])


def _edge_kernel(t, src, dst):
    return _edge_kernel_fn()(t, src, dst)


def _deg_body(dst_both, deg_out, acc, idx, ones, zbuf):
    c = lax.axis_index("c")
    s = lax.axis_index("s")
    zero16 = jnp.zeros((16,), jnp.float32)
    one16 = jnp.ones((16,), jnp.float32)

    def fill(q, carry):
        zbuf[pl.ds(q * 16, 16)] = zero16
        return carry

    lax.fori_loop(0, 40, fill, 0)
    for j in range(8):
        ones[pl.ds(j * 16, 16)] = one16
    pltpu.sync_copy(zbuf.at[pl.ds(0, 632)], acc.at[pl.ds(s * 632, 632)])
    plsc.subcore_barrier()

    ebase = c * _EPAD + s * _PER_TILE

    def body(b, carry):
        pltpu.sync_copy(dst_both.at[pl.ds(ebase + b * _KB, _KB)], idx)
        pltpu.sync_copy(ones, acc.at[idx], add=True)
        return carry

    lax.fori_loop(0, _NB, body, 0)
    plsc.subcore_barrier()
    pltpu.sync_copy(acc.at[pl.ds(s * 632, 632)], zbuf.at[pl.ds(0, 632)])
    pltpu.sync_copy(zbuf.at[pl.ds(0, 632)],
                    deg_out.at[pl.ds(c * _DEG_ROWS + s * 632, 632)])


@functools.lru_cache(maxsize=None)
def _deg_kernel_fn():
    mesh = plsc.VectorSubcoreMesh(
        core_axis_name="c", subcore_axis_name="s",
        num_cores=_CORES, num_subcores=_TILES)
    return pl.kernel(
        _deg_body,
        out_type=jax.ShapeDtypeStruct((_CORES * _DEG_ROWS,), jnp.float32),
        mesh=mesh,
        scratch_types=[
            pltpu.VMEM_SHARED((_DEG_ROWS,), jnp.float32),
            pltpu.VMEM((_KB,), jnp.int32),
            pltpu.VMEM((_KB,), jnp.float32),
            pltpu.VMEM((640,), jnp.float32),
        ])


def _deg_kernel(dst_both):
    return _deg_kernel_fn()(dst_both)


def _layer1(x, deg, W, b, interpret=False):
    def body(x_ref, deg_ref, w_ref, b_ref, t_ref, invs_ref):
        invs = lax.rsqrt(deg_ref[...] + 1.0)
        invs_ref[...] = invs
        g = jnp.dot(x_ref[...], w_ref[...],
                    preferred_element_type=jnp.float32, precision=_PREC)
        t_ref[...] = invs * (g + b_ref[...])

    return pl.pallas_call(
        body,
        grid=(_N // _R, 2),
        in_specs=[
            pl.BlockSpec((_R, 128), lambda i, h: (i, 0)),
            pl.BlockSpec((_R, 1), lambda i, h: (i, 0)),
            pl.BlockSpec((128, _H), lambda i, h: (0, h)),
            pl.BlockSpec((1, _H), lambda i, h: (0, h)),
        ],
        out_specs=[
            pl.BlockSpec((_R, _H), lambda i, h: (h * (_N // _R) + i, 0)),
            pl.BlockSpec((_R, 1), lambda i, h: (i, 0)),
        ],
        out_shape=[
            jax.ShapeDtypeStruct((2 * _N, _H), jnp.float32),
            jax.ShapeDtypeStruct((_N, 1), jnp.float32),
        ],
        interpret=interpret,
    )(x, deg, W, b.reshape(1, -1))


def _layer23(S3, t3, invs, W, b, interpret=False):
    def body(s_ref, t_ref, invs_ref, w_ref, b_ref, out_ref):
        invs = invs_ref[...]
        x0 = jnp.maximum(invs * (s_ref[0] + t_ref[0]), 0.0)
        x1 = jnp.maximum(invs * (s_ref[1] + t_ref[1]), 0.0)
        x = jnp.concatenate([x0, x1], axis=1)
        g = jnp.dot(x, w_ref[...],
                    preferred_element_type=jnp.float32, precision=_PREC)
        out_ref[...] = invs * (g + b_ref[...])

    return pl.pallas_call(
        body,
        grid=(_N // _R, 2),
        in_specs=[
            pl.BlockSpec((2, _R, 128), lambda i, h: (0, i, 0)),
            pl.BlockSpec((2, _R, 128), lambda i, h: (0, i, 0)),
            pl.BlockSpec((_R, 1), lambda i, h: (i, 0)),
            pl.BlockSpec((256, _H), lambda i, h: (0, h)),
            pl.BlockSpec((1, _H), lambda i, h: (0, h)),
        ],
        out_specs=pl.BlockSpec((_R, _H), lambda i, h: (h * (_N // _R) + i, 0)),
        out_shape=jax.ShapeDtypeStruct((2 * _N, _H), jnp.float32),
        interpret=interpret,
    )(S3, t3, invs, W, b.reshape(1, -1))


def _pool(S3, t3, invs, batch2, interpret=False):
    nblk = _N // _R

    def body(s_ref, t_ref, invs_ref, b_ref, out_ref, cnt_ref):
        i = pl.program_id(0)
        invs = invs_ref[...]
        x0 = jnp.maximum(invs * (s_ref[0] + t_ref[0]), 0.0)
        x1 = jnp.maximum(invs * (s_ref[1] + t_ref[1]), 0.0)
        h = jnp.concatenate([x0, x1], axis=1)
        ids = lax.broadcasted_iota(jnp.int32, (_R, _G), 1)
        oh = (b_ref[...] == ids).astype(jnp.float32)
        part = lax.dot_general(oh, h, (((0,), (0,)), ((), ())),
                               preferred_element_type=jnp.float32,
                               precision=_PREC)
        cpart = lax.dot_general(oh, jnp.ones((_R, 1), jnp.float32),
                                (((0,), (0,)), ((), ())),
                                preferred_element_type=jnp.float32,
                                precision=_PREC)

        @pl.when(i == 0)
        def _():
            out_ref[...] = jnp.zeros_like(out_ref)
            cnt_ref[...] = jnp.zeros_like(cnt_ref)

        out_ref[...] += part
        cnt_ref[...] += cpart

        @pl.when(i == nblk - 1)
        def _():
            out_ref[...] = out_ref[...] / jnp.maximum(cnt_ref[...], 1.0)

    return pl.pallas_call(
        body,
        grid=(nblk,),
        in_specs=[
            pl.BlockSpec((2, _R, 128), lambda i: (0, i, 0)),
            pl.BlockSpec((2, _R, 128), lambda i: (0, i, 0)),
            pl.BlockSpec((_R, 1), lambda i: (i, 0)),
            pl.BlockSpec((_R, 1), lambda i: (i, 0)),
        ],
        out_specs=pl.BlockSpec((_G, 256), lambda i: (0, 0)),
        out_shape=jax.ShapeDtypeStruct((_G, 256), jnp.float32),
        scratch_shapes=[pltpu.VMEM((_G, 1), jnp.float32)],
        interpret=interpret,
    )(S3, t3, invs, batch2)


def _classifier(p1, p2, W1a, W1b, b1, W2, b2, W3, b3, interpret=False):
    def body(p1_ref, p2_ref, w1a, w1b, b1_ref, w2_ref, b2_ref, w3_ref, b3_ref,
             out_ref):
        z = jnp.dot(p1_ref[...], w1a[...],
                    preferred_element_type=jnp.float32, precision=_PREC)
        z += jnp.dot(p2_ref[...], w1b[...],
                     preferred_element_type=jnp.float32, precision=_PREC)
        z = jnp.maximum(z + b1_ref[...], 0.0)
        z = jnp.maximum(
            jnp.dot(z, w2_ref[...], preferred_element_type=jnp.float32,
                    precision=_PREC) + b2_ref[...], 0.0)
        out_ref[...] = jnp.dot(
            z, w3_ref[...], preferred_element_type=jnp.float32,
            precision=_PREC) + b3_ref[...]

    return pl.pallas_call(
        body,
        out_shape=jax.ShapeDtypeStruct((_G, 86), jnp.float32),
        interpret=interpret,
    )(p1, p2, W1a, W1b, b1.reshape(1, -1), W2, b2.reshape(1, -1),
      W3, b3.reshape(1, -1))


def kernel(drug1_x, drug1_edge_index, drug1_batch,
           drug2_x, drug2_edge_index, drug2_batch,
           We1, be1, We2, be2, We3, be3,
           Wc1, bc1, Wc2, bc2, Wc3, bc3):
    pad = _EPAD - _E

    def prep(ei):
        src = jnp.concatenate([ei[0], jnp.zeros((pad,), jnp.int32)])
        dst = jnp.concatenate([ei[1], jnp.full((pad,), _N, jnp.int32)])
        return src, dst

    src1, dst1 = prep(drug1_edge_index)
    src2, dst2 = prep(drug2_edge_index)

    deg_flat = _deg_kernel(jnp.concatenate([dst1, dst2]))
    deg1 = deg_flat[0:_N].reshape(_N, 1)
    deg2 = deg_flat[_DEG_ROWS:_DEG_ROWS + _N].reshape(_N, 1)

    def encode(x, src, dst, deg, batch):
        t, invs = _layer1(x, deg, We1, be1)
        for W, b in ((We2, be2), (We3, be3)):
            S = _edge_kernel(t, src, dst)
            t = _layer23(S.reshape(2, _ACC_ROWS, _H), t.reshape(2, _N, _H),
                         invs, W, b)
        S = _edge_kernel(t, src, dst)
        return _pool(S.reshape(2, _ACC_ROWS, _H), t.reshape(2, _N, _H),
                     invs, batch.reshape(_N, 1))

    p1 = encode(drug1_x, src1, dst1, deg1, drug1_batch)
    p2 = encode(drug2_x, src2, dst2, deg2, drug2_batch)
    return _classifier(p1, p2, Wc1[:256], Wc1[256:], bc1, Wc2, bc2, Wc3, bc3)
